# baseline (speedup 1.0000x reference)
import functools

import jax
import jax.numpy as jnp
from jax import lax
from jax.experimental import pallas as pl
from jax.experimental.pallas import tpu as pltpu
from jax.experimental.pallas import tpu_sc as plsc

NUM_RULES = 100000
RULE_DIM = 64
BATCH = 4096
NUM_ACTIVE = 26

NC = 2
NS = 16
NW = NC * NS

TOTAL = BATCH * NUM_ACTIVE
B_PER_W = TOTAL // NW
CHUNK = 128
NCHUNK = B_PER_W // CHUNK
NSLOT = 15
DEPTH = 7


@functools.partial(
    pl.kernel,
    mesh=plsc.VectorSubcoreMesh(core_axis_name="c", subcore_axis_name="s"),
    out_type=jax.ShapeDtypeStruct((TOTAL, RULE_DIM), jnp.float32),
    scratch_types=[
        pltpu.VMEM((NCHUNK, CHUNK), jnp.int32),
        pltpu.VMEM((NSLOT, CHUNK, RULE_DIM), jnp.float32),
        pltpu.SemaphoreType.DMA,
        pltpu.SemaphoreType.DMA,
    ],
    compiler_params=pltpu.CompilerParams(use_tc_tiling_on_sc=False),
)
def _gather(idx_hbm, table_hbm, out_hbm, idx_v, rows_v, sem_in, sem_out):
    wid = lax.axis_index("s") * NC + lax.axis_index("c")
    base = wid * B_PER_W
    pltpu.sync_copy(idx_hbm.at[wid], idx_v)
    gets = [None] * NCHUNK
    puts = [None] * NCHUNK
    for g in range(NCHUNK + DEPTH):
        if g < NCHUNK:
            if g >= NSLOT:
                puts[g - NSLOT].wait()
            gets[g] = pltpu.async_copy(
                table_hbm.at[idx_v.at[g]], rows_v.at[g % NSLOT], sem_in
            )
        if g >= DEPTH:
            d = g - DEPTH
            gets[d].wait()
            puts[d] = pltpu.async_copy(
                rows_v.at[d % NSLOT],
                out_hbm.at[pl.ds(base + d * CHUNK, CHUNK)],
                sem_out,
            )
    for d in range(NCHUNK - NSLOT, NCHUNK):
        puts[d].wait()


def kernel(rule_indices, rules):
    idx = rule_indices.astype(jnp.int32).reshape(NW, NCHUNK, CHUNK)
    out = _gather(idx, rules)
    return out.reshape(BATCH, NUM_ACTIVE, RULE_DIM)


# speedup vs baseline: 1.9256x; 1.9256x over previous
import functools

import jax
import jax.numpy as jnp
from jax import lax
from jax.experimental import pallas as pl
from jax.experimental.pallas import tpu as pltpu
from jax.experimental.pallas import tpu_sc as plsc

NUM_RULES = 100000
RULE_DIM = 64
BATCH = 4096
NUM_ACTIVE = 26

NC = 2
NS = 16
ROUNDS = RULE_DIM // (NC * NS)
LANES = 16


@functools.partial(
    pl.kernel,
    mesh=plsc.VectorSubcoreMesh(core_axis_name="c", subcore_axis_name="s"),
    out_type=jax.ShapeDtypeStruct((NUM_ACTIVE, RULE_DIM, BATCH), jnp.float32),
    scratch_types=[
        pltpu.VMEM((NUM_RULES,), jnp.float32),
        pltpu.VMEM((2, BATCH), jnp.int32),
        pltpu.VMEM((2, BATCH), jnp.float32),
        pltpu.SemaphoreType.DMA,
        pltpu.SemaphoreType.DMA,
        pltpu.SemaphoreType.DMA,
    ],
    compiler_params=pltpu.CompilerParams(needs_layout_passes=False),
)
def _tgather(t_hbm, i_hbm, out_hbm, row_v, idx_v, outb_v, sem_row, sem_in,
             sem_out):
    c = lax.axis_index("c")
    s = lax.axis_index("s")
    for r in range(ROUNDS):
        d = c * (NS * ROUNDS) + r * NS + s
        pltpu.async_copy(t_hbm.at[d], row_v, sem_row).wait()
        pltpu.async_copy(i_hbm.at[0], idx_v.at[0], sem_in)

        def a_body(a, _):
            buf = lax.rem(a, 2)
            pltpu.make_async_copy(i_hbm.at[a], idx_v.at[buf], sem_in).wait()

            @pl.when(a + 1 < NUM_ACTIVE)
            def _():
                pltpu.async_copy(i_hbm.at[a + 1], idx_v.at[1 - buf], sem_in)

            @pl.when(a >= 2)
            def _():
                pltpu.make_async_copy(
                    outb_v.at[buf], out_hbm.at[a - 2, d], sem_out
                ).wait()

            def g_body(k):
                iv = idx_v[buf, pl.ds(k, LANES)]
                outb_v[buf, pl.ds(k, LANES)] = plsc.load_gather(row_v, [iv])

            plsc.parallel_loop(0, BATCH, LANES, unroll=8)(g_body)
            pltpu.async_copy(outb_v.at[buf], out_hbm.at[a, d], sem_out)
            return ()

        lax.fori_loop(0, NUM_ACTIVE, a_body, ())
        for a in (NUM_ACTIVE - 2, NUM_ACTIVE - 1):
            pltpu.make_async_copy(
                outb_v.at[a % 2], out_hbm.at[a, d], sem_out
            ).wait()


def kernel(rule_indices, rules):
    t = rules.T
    idx = rule_indices.T.astype(jnp.int32)
    out = _tgather(t, idx)
    return jnp.transpose(out, (2, 0, 1))
